# Initial kernel scaffold; baseline (speedup 1.0000x reference)
import jax
import jax.numpy as jnp
from jax.experimental import pallas as pl


def kernel(x, emb, w1, b1, w2, b2, wg, bg, wl1, bl1, wl2, bl2, wc1, bc1, wc2, bc2):
    raise NotImplementedError("write your pallas kernel here")


# speedup vs baseline: 1.8087x; 1.8087x over previous
import functools

import jax
import jax.numpy as jnp
from jax import lax
from jax.experimental import pallas as pl
from jax.experimental.pallas import tpu as pltpu
from jax.experimental.pallas import tpu_sc as plsc

V = 100000
D = 64
SL = 50
L = 200
B = 16
K = 5
H1 = 250
H2 = 100
HG = 100
HL = 50

C = B * L
CP = 3328
TOT = CP * SL
NW = 32
PW = TOT // NW
CH = 104
NCH = PW // CH

H1P = 256
CB = 256

F32 = jnp.float32
BF16 = jnp.bfloat16


def _sc_gather(emb_bf, xf):
    mesh = plsc.VectorSubcoreMesh(core_axis_name="c", subcore_axis_name="s")

    @functools.partial(
        pl.kernel,
        out_type=jax.ShapeDtypeStruct((TOT, D), BF16),
        mesh=mesh,
        compiler_params=pltpu.CompilerParams(use_tc_tiling_on_sc=False),
        scratch_types=[
            pltpu.VMEM((PW,), jnp.int32),
            pltpu.VMEM((CH, D), BF16),
            pltpu.SemaphoreType.DMA,
        ],
    )
    def gk(emb_h, xf_h, out_h, idx_v, rows_v, sem):
        wid = lax.axis_index("s") * 2 + lax.axis_index("c")
        base = wid * PW
        pltpu.sync_copy(xf_h.at[pl.ds(base, PW)], idx_v)

        def body(j, carry):
            off = j * CH
            pltpu.async_copy(
                emb_h.at[idx_v.at[pl.ds(off, CH)]], rows_v, sem
            ).wait()
            pltpu.sync_copy(rows_v, out_h.at[pl.ds(base + off, CH)])
            return carry

        lax.fori_loop(0, NCH, body, 0)

    return gk(emb_bf, xf)


def _conv1_body(e_ref, w_ref, b_ref, h_ref):
    w = w_ref[...]
    m = jnp.full((CB, H1P), -jnp.inf, dtype=F32)
    for t in range(SL - 2):
        lhs = e_ref[:, t * D:(t + 3) * D]
        y = jnp.dot(lhs, w, preferred_element_type=F32)
        m = jnp.maximum(m, y)
    h_ref[...] = jnp.maximum(m + b_ref[...], 0.0)


def _topk_body(logp_ref, z_ref):
    v = logp_ref[...]
    iot = lax.broadcasted_iota(jnp.int32, (B, L), 1)
    zm = jnp.zeros((B, L), dtype=F32)
    for _ in range(K):
        mval = jnp.max(v, axis=1, keepdims=True)
        idx = jnp.min(jnp.where(v == mval, iot, jnp.int32(2 ** 30)),
                      axis=1, keepdims=True)
        hit = iot == idx
        zm = jnp.where(hit, 1.0, zm)
        v = jnp.where(hit, -jnp.inf, v)
    z_ref[...] = zm


def _conv1d(x, w, b, pad=0):
    y = lax.conv_general_dilated(x, w, window_strides=(1,), padding=[(pad, pad)],
                                 dimension_numbers=('NCH', 'OIH', 'NCH'))
    return y + b[None, :, None]


def kernel(x, emb, w1, b1, w2, b2, wg, bg, wl1, bl1, wl2, bl2,
           wc1, bc1, wc2, bc2):
    xf = jnp.concatenate(
        [x.reshape(-1), jnp.zeros((TOT - C * SL,), dtype=jnp.int32)])
    e = _sc_gather(emb.astype(BF16), xf)
    e2 = e.reshape(CP, SL * D)

    w1c = jnp.pad(jnp.transpose(w1, (2, 1, 0)).reshape(3 * D, H1),
                  ((0, 0), (0, H1P - H1))).astype(BF16)
    b1p = jnp.pad(b1, (0, H1P - H1)).reshape(1, H1P)
    h = pl.pallas_call(
        _conv1_body,
        grid=(CP // CB,),
        in_specs=[
            pl.BlockSpec((CB, SL * D), lambda i: (i, 0)),
            pl.BlockSpec((3 * D, H1P), lambda i: (0, 0)),
            pl.BlockSpec((1, H1P), lambda i: (0, 0)),
        ],
        out_specs=pl.BlockSpec((CB, H1P), lambda i: (i, 0)),
        out_shape=jax.ShapeDtypeStruct((CP, H1P), F32),
    )(e2, w1c, b1p)

    h3 = h[:C, :H1].reshape(B, L, H1).transpose(0, 2, 1)
    h2 = jax.nn.relu(_conv1d(h3, w2, b2, pad=1))
    g = jnp.max(h2, axis=-1)
    g = jax.nn.relu(g @ wg.T + bg)
    loc = jax.nn.relu(_conv1d(h3, wl1, bl1, pad=1))
    loc = jax.nn.relu(_conv1d(loc, wl2, bl2, pad=1))
    gexp = jnp.broadcast_to(g[:, None, :], (B, L, HG))
    cat = jnp.concatenate([gexp, loc.transpose(0, 2, 1)], axis=-1)
    c = cat.transpose(0, 2, 1)
    c = jax.nn.relu(_conv1d(c, wc1, bc1))
    c = _conv1d(c, wc2, bc2)
    logp = jax.nn.log_softmax(c[:, 0, :], axis=-1)

    z = pl.pallas_call(
        _topk_body,
        in_specs=[pl.BlockSpec((B, L), lambda: (0, 0))],
        out_specs=pl.BlockSpec((B, L), lambda: (0, 0)),
        out_shape=jax.ShapeDtypeStruct((B, L), F32),
    )(logp)

    return logp, z


# speedup vs baseline: 1.8998x; 1.0504x over previous
import functools

import jax
import jax.numpy as jnp
from jax import lax
from jax.experimental import pallas as pl
from jax.experimental.pallas import tpu as pltpu
from jax.experimental.pallas import tpu_sc as plsc

V = 100000
D = 64
SL = 50
L = 200
B = 16
K = 5
H1 = 250
H2 = 100
HG = 100
HL = 50

C = B * L
CP = 3328
TOT = CP * SL
NW = 32
PW = TOT // NW
CH = 104
NCH = PW // CH

H1P = 256
CB = 256

F32 = jnp.float32
BF16 = jnp.bfloat16


def _sc_gather(emb_bf, xf):
    mesh = plsc.VectorSubcoreMesh(core_axis_name="c", subcore_axis_name="s")

    @functools.partial(
        pl.kernel,
        out_type=jax.ShapeDtypeStruct((TOT, D), BF16),
        mesh=mesh,
        compiler_params=pltpu.CompilerParams(use_tc_tiling_on_sc=False),
        scratch_types=[
            pltpu.VMEM((PW,), jnp.int32),
            pltpu.VMEM((CH, D), BF16),
            pltpu.VMEM((CH, D), BF16),
            pltpu.SemaphoreType.DMA,
            pltpu.SemaphoreType.DMA,
        ],
    )
    def gk(emb_h, xf_h, out_h, idx_v, r0, r1, s0, s1):
        wid = lax.axis_index("s") * 2 + lax.axis_index("c")
        base = wid * PW
        pltpu.sync_copy(xf_h.at[pl.ds(base, PW)], idx_v)

        def mk(j, r, s):
            return pltpu.make_async_copy(
                emb_h.at[idx_v.at[pl.ds(j * CH, CH)]], r, s)

        mk(0, r0, s0).start()

        def body(i, carry):
            j0 = 2 * i
            mk(j0 + 1, r1, s1).start()
            mk(j0, r0, s0).wait()
            pltpu.sync_copy(r0, out_h.at[pl.ds(base + j0 * CH, CH)])

            @pl.when(i < NCH // 2 - 1)
            def _():
                mk(j0 + 2, r0, s0).start()

            mk(j0 + 1, r1, s1).wait()
            pltpu.sync_copy(r1, out_h.at[pl.ds(base + (j0 + 1) * CH, CH)])
            return carry

        lax.fori_loop(0, NCH // 2, body, 0)

    return gk(emb_bf, xf)


def _conv1_body(e_ref, w_ref, b_ref, h_ref):
    w = w_ref[...]
    m = jnp.full((CB, H1P), -jnp.inf, dtype=F32)
    for t in range(SL - 2):
        lhs = e_ref[:, t * D:(t + 3) * D]
        y = jnp.dot(lhs, w, preferred_element_type=F32)
        m = jnp.maximum(m, y)
    h_ref[...] = jnp.maximum(m + b_ref[...], 0.0)


def _topk_body(logp_ref, z_ref):
    v = logp_ref[...]
    iot = lax.broadcasted_iota(jnp.int32, (B, L), 1)
    zm = jnp.zeros((B, L), dtype=F32)
    for _ in range(K):
        mval = jnp.max(v, axis=1, keepdims=True)
        idx = jnp.min(jnp.where(v == mval, iot, jnp.int32(2 ** 30)),
                      axis=1, keepdims=True)
        hit = iot == idx
        zm = jnp.where(hit, 1.0, zm)
        v = jnp.where(hit, -jnp.inf, v)
    z_ref[...] = zm


def _conv1d(x, w, b, pad=0):
    y = lax.conv_general_dilated(x, w, window_strides=(1,), padding=[(pad, pad)],
                                 dimension_numbers=('NCH', 'OIH', 'NCH'))
    return y + b[None, :, None]


def kernel(x, emb, w1, b1, w2, b2, wg, bg, wl1, bl1, wl2, bl2,
           wc1, bc1, wc2, bc2):
    xf = jnp.concatenate(
        [x.reshape(-1), jnp.zeros((TOT - C * SL,), dtype=jnp.int32)])
    e = _sc_gather(emb.astype(BF16), xf)
    e2 = e.reshape(CP, SL * D)

    w1c = jnp.pad(jnp.transpose(w1, (2, 1, 0)).reshape(3 * D, H1),
                  ((0, 0), (0, H1P - H1))).astype(BF16)
    b1p = jnp.pad(b1, (0, H1P - H1)).reshape(1, H1P)
    h = pl.pallas_call(
        _conv1_body,
        grid=(CP // CB,),
        in_specs=[
            pl.BlockSpec((CB, SL * D), lambda i: (i, 0)),
            pl.BlockSpec((3 * D, H1P), lambda i: (0, 0)),
            pl.BlockSpec((1, H1P), lambda i: (0, 0)),
        ],
        out_specs=pl.BlockSpec((CB, H1P), lambda i: (i, 0)),
        out_shape=jax.ShapeDtypeStruct((CP, H1P), F32),
    )(e2, w1c, b1p)

    h3 = h[:C, :H1].reshape(B, L, H1).transpose(0, 2, 1)
    h2 = jax.nn.relu(_conv1d(h3, w2, b2, pad=1))
    g = jnp.max(h2, axis=-1)
    g = jax.nn.relu(g @ wg.T + bg)
    loc = jax.nn.relu(_conv1d(h3, wl1, bl1, pad=1))
    loc = jax.nn.relu(_conv1d(loc, wl2, bl2, pad=1))
    gexp = jnp.broadcast_to(g[:, None, :], (B, L, HG))
    cat = jnp.concatenate([gexp, loc.transpose(0, 2, 1)], axis=-1)
    c = cat.transpose(0, 2, 1)
    c = jax.nn.relu(_conv1d(c, wc1, bc1))
    c = _conv1d(c, wc2, bc2)
    logp = jax.nn.log_softmax(c[:, 0, :], axis=-1)

    z = pl.pallas_call(
        _topk_body,
        in_specs=[pl.BlockSpec((B, L), lambda: (0, 0))],
        out_specs=pl.BlockSpec((B, L), lambda: (0, 0)),
        out_shape=jax.ShapeDtypeStruct((B, L), F32),
    )(logp)

    return logp, z


# speedup vs baseline: 1.9391x; 1.0207x over previous
import functools

import jax
import jax.numpy as jnp
from jax import lax
from jax.experimental import pallas as pl
from jax.experimental.pallas import tpu as pltpu
from jax.experimental.pallas import tpu_sc as plsc

V = 100000
D = 64
SL = 50
L = 200
B = 16
K = 5
H1 = 250
H2 = 100
HG = 100
HL = 50

C = B * L
CP = 3328
TOT = CP * SL
NW = 32
PW = TOT // NW
CH = 104
NCH = PW // CH

H1P = 256
CB = 256

F32 = jnp.float32
BF16 = jnp.bfloat16


GCH = 520
NG = PW // GCH


def _sc_gather(emb_bf, xf):
    mesh = plsc.VectorSubcoreMesh(core_axis_name="c", subcore_axis_name="s")

    @functools.partial(
        pl.kernel,
        out_type=jax.ShapeDtypeStruct((TOT, D), BF16),
        mesh=mesh,
        compiler_params=pltpu.CompilerParams(use_tc_tiling_on_sc=False),
        scratch_types=[
            pltpu.VMEM((PW,), jnp.int32),
            pltpu.VMEM((GCH, D), BF16),
            pltpu.VMEM((GCH, D), BF16),
            pltpu.SemaphoreType.DMA,
            pltpu.SemaphoreType.DMA,
        ],
    )
    def gk(emb_h, xf_h, out_h, idx_v, r0, r1, s0, s1):
        wid = lax.axis_index("s") * 2 + lax.axis_index("c")
        base = wid * PW
        pltpu.sync_copy(xf_h.at[pl.ds(base, PW)], idx_v)

        bufs = (r0, r1)
        sems = (s0, s1)

        def mk(g, r, s):
            return pltpu.make_async_copy(
                emb_h.at[idx_v.at[pl.ds(g * GCH, GCH)]], r, s)

        mk(0, r0, s0).start()
        for g in range(NG):
            r, s = bufs[g % 2], sems[g % 2]
            if g + 1 < NG:
                mk(g + 1, bufs[(g + 1) % 2], sems[(g + 1) % 2]).start()
            mk(g, r, s).wait()
            pltpu.sync_copy(r, out_h.at[pl.ds(base + g * GCH, GCH)])

    return gk(emb_bf, xf)


def _conv1_body(e_ref, w_ref, b_ref, h_ref):
    w = w_ref[...]
    m = jnp.full((CB, H1P), -jnp.inf, dtype=F32)
    for t in range(SL - 2):
        lhs = e_ref[:, t * D:(t + 3) * D]
        y = jnp.dot(lhs, w, preferred_element_type=F32)
        m = jnp.maximum(m, y)
    h_ref[...] = jnp.maximum(m + b_ref[...], 0.0)


def _topk_body(logp_ref, z_ref):
    v = logp_ref[...]
    iot = lax.broadcasted_iota(jnp.int32, (B, L), 1)
    zm = jnp.zeros((B, L), dtype=F32)
    for _ in range(K):
        mval = jnp.max(v, axis=1, keepdims=True)
        idx = jnp.min(jnp.where(v == mval, iot, jnp.int32(2 ** 30)),
                      axis=1, keepdims=True)
        hit = iot == idx
        zm = jnp.where(hit, 1.0, zm)
        v = jnp.where(hit, -jnp.inf, v)
    z_ref[...] = zm


def _conv1d(x, w, b, pad=0):
    y = lax.conv_general_dilated(x, w, window_strides=(1,), padding=[(pad, pad)],
                                 dimension_numbers=('NCH', 'OIH', 'NCH'))
    return y + b[None, :, None]


def kernel(x, emb, w1, b1, w2, b2, wg, bg, wl1, bl1, wl2, bl2,
           wc1, bc1, wc2, bc2):
    xf = jnp.concatenate(
        [x.reshape(-1), jnp.zeros((TOT - C * SL,), dtype=jnp.int32)])
    e = _sc_gather(emb.astype(BF16), xf)
    e2 = e.reshape(CP, SL * D)

    w1c = jnp.pad(jnp.transpose(w1, (2, 1, 0)).reshape(3 * D, H1),
                  ((0, 0), (0, H1P - H1))).astype(BF16)
    b1p = jnp.pad(b1, (0, H1P - H1)).reshape(1, H1P)
    h = pl.pallas_call(
        _conv1_body,
        grid=(CP // CB,),
        in_specs=[
            pl.BlockSpec((CB, SL * D), lambda i: (i, 0)),
            pl.BlockSpec((3 * D, H1P), lambda i: (0, 0)),
            pl.BlockSpec((1, H1P), lambda i: (0, 0)),
        ],
        out_specs=pl.BlockSpec((CB, H1P), lambda i: (i, 0)),
        out_shape=jax.ShapeDtypeStruct((CP, H1P), F32),
    )(e2, w1c, b1p)

    h3 = h[:C, :H1].reshape(B, L, H1).transpose(0, 2, 1)
    h2 = jax.nn.relu(_conv1d(h3, w2, b2, pad=1))
    g = jnp.max(h2, axis=-1)
    g = jax.nn.relu(g @ wg.T + bg)
    loc = jax.nn.relu(_conv1d(h3, wl1, bl1, pad=1))
    loc = jax.nn.relu(_conv1d(loc, wl2, bl2, pad=1))
    gexp = jnp.broadcast_to(g[:, None, :], (B, L, HG))
    cat = jnp.concatenate([gexp, loc.transpose(0, 2, 1)], axis=-1)
    c = cat.transpose(0, 2, 1)
    c = jax.nn.relu(_conv1d(c, wc1, bc1))
    c = _conv1d(c, wc2, bc2)
    logp = jax.nn.log_softmax(c[:, 0, :], axis=-1)

    z = pl.pallas_call(
        _topk_body,
        in_specs=[pl.BlockSpec((B, L), lambda: (0, 0))],
        out_specs=pl.BlockSpec((B, L), lambda: (0, 0)),
        out_shape=jax.ShapeDtypeStruct((B, L), F32),
    )(logp)

    return logp, z
